# baseline (speedup 1.0000x reference)
import functools

import jax
import jax.numpy as jnp
from jax import lax
from jax.experimental import pallas as pl
from jax.experimental.pallas import tpu as pltpu
from jax.experimental.pallas import tpu_sc as plsc

_S = 128
_B = 8
_FLAT = _S * _S
_PAD = 64
_INV_COUNT = 1.0 / (_B * (_S - 2))


def _make_kernel():
    mesh = plsc.VectorSubcoreMesh(
        core_axis_name="c", subcore_axis_name="s", num_cores=1
    )

    @functools.partial(
        pl.kernel,
        mesh=mesh,
        out_type=jax.ShapeDtypeStruct((16, 16), jnp.float32),
        compiler_params=pltpu.CompilerParams(needs_layout_passes=False),
        scratch_types=[
            pltpu.VMEM((_FLAT + _PAD,), jnp.float32),
            pltpu.VMEM((16,), jnp.float32),
        ],
    )
    def diag_std_kernel(attn_hbm, out_hbm, buf, part_v):
        s = lax.axis_index("s")
        batch = s >> 1
        half = s & 1

        def _copy(nrows):
            def f():
                pltpu.sync_copy(
                    attn_hbm.at[batch, pl.ds(0, nrows * _S)],
                    buf.at[pl.ds(0, nrows * _S)],
                )
            return f

        lax.switch(half, [_copy(127), _copy(95)])

        lane = lax.iota(jnp.int32, 16)

        def sel(a, b):
            return jnp.where(half == 0, a, b)

        d0s = [sel(1, 33), sel(17, 49), sel(97, 65), sel(113, 81)]
        dvs = [d0 + lane for d0 in d0s]
        nchunk_seq = [4, 2, 1]
        nblocks = [sel(16, 32), sel(40, 8), 8]

        def unit(base, dv, i, sacc, qacc):
            x = buf[pl.ds(base, 16)]
            m = (dv + i) < _S
            x = jnp.where(m, x, 0.0)
            return sacc + x, qacc + x * x

        def unit_nm(base, sacc, qacc):
            x = buf[pl.ds(base, 16)]
            return sacc + x, qacc + x * x

        zero = jnp.zeros((16,), jnp.float32)
        accs = [(zero, zero)] * 4
        row0 = 0
        for phase in range(3):
            nchunks = nchunk_seq[phase]

            nm_chunks = 2 - phase

            def body(j, carry, row0=row0, nchunks=nchunks, nm=nm_chunks):
                out = list(carry)
                i = row0 + j * 2
                for k in range(nchunks):
                    sc_, qc_ = out[2 * k], out[2 * k + 1]
                    base = d0s[k] + 129 * i
                    if k < nm:
                        sc_, qc_ = unit_nm(base, sc_, qc_)
                        sc_, qc_ = unit_nm(base + 129, sc_, qc_)
                    else:
                        sc_, qc_ = unit(base, dvs[k], i, sc_, qc_)
                        sc_, qc_ = unit(base + 129, dvs[k], i + 1, sc_, qc_)
                    out[2 * k], out[2 * k + 1] = sc_, qc_
                return tuple(out)

            flat_accs = tuple(x for pair in accs[:nchunks] for x in pair)
            flat_accs = lax.fori_loop(0, nblocks[phase], body, flat_accs)
            for k in range(nchunks):
                accs[k] = (flat_accs[2 * k], flat_accs[2 * k + 1])
            row0 = row0 + nblocks[phase] * 2

        nfs = [(_S - dv).astype(jnp.float32) for dv in dvs]
        var4 = [
            jnp.maximum((q - s_ * s_ / nf) / (nf - 1.0), 0.0)
            for (s_, q), nf in zip(accs, nfs)
        ]
        ys = tuple((v + 1.0) * 0.5 for v in var4)

        def newton(_, ys):
            return tuple(0.5 * (y + v / y) for y, v in zip(ys, var4))

        ys = lax.fori_loop(0, 10, newton, ys)

        partial = zero
        for k in range(4):
            partial = partial + jnp.where(
                dvs[k] <= _S - 2, ys[k] * nfs[k] * 0.2, 0.0
            )
        part_v[...] = partial * _INV_COUNT
        pltpu.sync_copy(part_v, out_hbm.at[s])

    return diag_std_kernel


_diag_std = _make_kernel()


def kernel(attn):
    flat = attn.reshape(_B, _FLAT)
    out = _diag_std(flat)
    return jnp.sum(out)


# speedup vs baseline: 1.0023x; 1.0023x over previous
import functools

import jax
import jax.numpy as jnp
from jax import lax
from jax.experimental import pallas as pl
from jax.experimental.pallas import tpu as pltpu
from jax.experimental.pallas import tpu_sc as plsc

_S = 128
_B = 8
_FLAT = _S * _S
_PAD = 64
_INV_COUNT = 1.0 / (_B * (_S - 2))


def _make_kernel():
    mesh = plsc.VectorSubcoreMesh(
        core_axis_name="c", subcore_axis_name="s", num_cores=1
    )

    @functools.partial(
        pl.kernel,
        mesh=mesh,
        out_type=jax.ShapeDtypeStruct((16, 16), jnp.float32),
        compiler_params=pltpu.CompilerParams(needs_layout_passes=False),
        scratch_types=[
            pltpu.VMEM((_FLAT + _PAD,), jnp.float32),
            pltpu.VMEM((16,), jnp.float32),
        ],
    )
    def diag_std_kernel(attn_hbm, out_hbm, buf, part_v):
        s = lax.axis_index("s")
        batch = s >> 1
        half = s & 1

        def _copy(nrows):
            def f():
                pltpu.sync_copy(
                    attn_hbm.at[batch, pl.ds(0, nrows * _S)],
                    buf.at[pl.ds(0, nrows * _S)],
                )
            return f

        lax.switch(half, [_copy(127), _copy(95)])

        lane = lax.iota(jnp.int32, 16)

        def sel(a, b):
            return jnp.where(half == 0, a, b)

        d0s = [sel(1, 33), sel(17, 49), sel(97, 65), sel(113, 81)]
        dvs = [d0 + lane for d0 in d0s]
        nchunk_seq = [4, 2, 1]
        nblocks = [sel(16, 32), sel(40, 8), 8]

        def unit(base, dv, i, sacc, qacc):
            x = buf[pl.ds(base, 16)]
            m = (dv + i) < _S
            x = jnp.where(m, x, 0.0)
            return sacc + x, qacc + x * x

        zero = jnp.zeros((16,), jnp.float32)
        accs = [(zero, zero)] * 4
        row0 = 0
        for phase in range(3):
            nchunks = nchunk_seq[phase]

            def body(j, carry, row0=row0, nchunks=nchunks):
                out = list(carry)
                i = row0 + j * 2
                for k in range(nchunks):
                    sc_, qc_ = out[2 * k], out[2 * k + 1]
                    base = d0s[k] + 129 * i
                    sc_, qc_ = unit(base, dvs[k], i, sc_, qc_)
                    sc_, qc_ = unit(base + 129, dvs[k], i + 1, sc_, qc_)
                    out[2 * k], out[2 * k + 1] = sc_, qc_
                return tuple(out)

            flat_accs = tuple(x for pair in accs[:nchunks] for x in pair)
            flat_accs = lax.fori_loop(0, nblocks[phase], body, flat_accs)
            for k in range(nchunks):
                accs[k] = (flat_accs[2 * k], flat_accs[2 * k + 1])
            row0 = row0 + nblocks[phase] * 2

        nfs = [(_S - dv).astype(jnp.float32) for dv in dvs]
        var4 = [
            jnp.maximum((q - s_ * s_ / nf) / (nf - 1.0), 0.0)
            for (s_, q), nf in zip(accs, nfs)
        ]
        ys = tuple((v + 1.0) * 0.5 for v in var4)

        def newton(_, ys):
            return tuple(0.5 * (y + v / y) for y, v in zip(ys, var4))

        ys = lax.fori_loop(0, 12, newton, ys)

        partial = zero
        for k in range(4):
            partial = partial + jnp.where(
                dvs[k] <= _S - 2, ys[k] * nfs[k] * 0.2, 0.0
            )
        part_v[...] = partial * _INV_COUNT
        pltpu.sync_copy(part_v, out_hbm.at[s])

    return diag_std_kernel


_diag_std = _make_kernel()


def kernel(attn):
    flat = attn.reshape(_B, _FLAT)
    out = _diag_std(flat)
    return jnp.sum(out)
